# baseline (speedup 1.0000x reference)
import functools

import jax
import jax.numpy as jnp
from jax import lax
from jax.experimental import pallas as pl
from jax.experimental.pallas import tpu as pltpu
from jax.experimental.pallas import tpu_sc as plsc

N = 10000
E = 320000
D = 128
LD = 32
RD = 32
NL = 1024
RB = 4096

NC = 2
NS = 16
NW = NC * NS

NP = 10240
TAB_ROWS = NP + RB
CHUNK = 32
ROWS = 2 * CHUNK
CH = 320
NB = 4
EPAD = NW * CH * CHUNK
ZROWS = NP // NS

BLK = 1024
GRID_N = NP // BLK
GRID_TAB = TAB_ROWS // BLK


def _ln_relu(pre, g, b):
    h = jnp.maximum(pre, 0.0)
    mu = jnp.mean(h, axis=-1, keepdims=True)
    var = jnp.mean((h - mu) * (h - mu), axis=-1, keepdims=True)
    return (h - mu) * lax.rsqrt(var + 1e-5) * g + b


def _nmap(i):
    return (jnp.minimum(i, GRID_N - 1), 0)


def _rmap(i):
    return (jnp.maximum(i - GRID_N, 0), 0)


def _tc_first_body(x_ref, lab_ref, lemb_ref, wtop_ref, wbot_ref, bin_ref,
                   wn_ref, ws_ref, bs_ref, rel_ref, wr_ref,
                   tab_ref, s_ref, t_ref):
    i = pl.program_id(0)

    @pl.when(i == 0)
    def _():
        t_ref[...] = lemb_ref[...] @ wbot_ref[...]

    @pl.when(i < GRID_N)
    def _():
        onehot = jnp.where(
            lab_ref[...] == lax.broadcasted_iota(jnp.int32, (1, NL), 1),
            1.0, 0.0)
        h = jnp.maximum(
            x_ref[...] @ wtop_ref[...] + onehot @ t_ref[...] + bin_ref[...],
            0.0)
        tab_ref[...] = h @ wn_ref[...]
        s_ref[...] = h @ ws_ref[...] + bs_ref[...]

    @pl.when(i >= GRID_N)
    def _():
        tab_ref[...] = rel_ref[...] @ wr_ref[...]


def _tc_first(x, lab, lemb, wtop, wbot, bin2, wn, ws, bs2, relemb, wr):
    return pl.pallas_call(
        _tc_first_body,
        grid=(GRID_TAB,),
        in_specs=[
            pl.BlockSpec((BLK, D), _nmap),
            pl.BlockSpec((BLK, 1), _nmap),
            pl.BlockSpec((NL, LD), lambda i: (0, 0)),
            pl.BlockSpec((D, D), lambda i: (0, 0)),
            pl.BlockSpec((LD, D), lambda i: (0, 0)),
            pl.BlockSpec((1, D), lambda i: (0, 0)),
            pl.BlockSpec((D, D), lambda i: (0, 0)),
            pl.BlockSpec((D, D), lambda i: (0, 0)),
            pl.BlockSpec((1, D), lambda i: (0, 0)),
            pl.BlockSpec((BLK, RD), _rmap),
            pl.BlockSpec((RD, D), lambda i: (0, 0)),
        ],
        out_specs=[
            pl.BlockSpec((BLK, D), lambda i: (i, 0)),
            pl.BlockSpec((BLK, D), _nmap),
        ],
        out_shape=[
            jax.ShapeDtypeStruct((TAB_ROWS, D), jnp.float32),
            jax.ShapeDtypeStruct((NP, D), jnp.float32),
        ],
        scratch_shapes=[pltpu.VMEM((NL, D), jnp.float32)],
    )(x, lab, lemb, wtop, wbot, bin2, wn, ws, bs2, relemb, wr)


def _tc_mid_body(s_ref, acc_ref, cnt_ref, g_ref, b_ref, wn_ref, ws_ref,
                 bs_ref, rel_ref, wr_ref, tab_ref, so_ref):
    i = pl.program_id(0)

    @pl.when(i < GRID_N)
    def _():
        agg = acc_ref[0, :, :] + acc_ref[1, :, :]
        agg = agg / jnp.maximum(cnt_ref[...], 1.0)
        h = _ln_relu(s_ref[...] + agg, g_ref[...], b_ref[...])
        tab_ref[...] = h @ wn_ref[...]
        so_ref[...] = h @ ws_ref[...] + bs_ref[...]

    @pl.when(i >= GRID_N)
    def _():
        tab_ref[...] = rel_ref[...] @ wr_ref[...]


def _tc_mid(s, acc, cnt, g2, b2, wn, ws, bs2, relemb, wr):
    return pl.pallas_call(
        _tc_mid_body,
        grid=(GRID_TAB,),
        in_specs=[
            pl.BlockSpec((BLK, D), _nmap),
            pl.BlockSpec((2, BLK, D), lambda i: (0, *_nmap(i))),
            pl.BlockSpec((BLK, 1), _nmap),
            pl.BlockSpec((1, D), lambda i: (0, 0)),
            pl.BlockSpec((1, D), lambda i: (0, 0)),
            pl.BlockSpec((D, D), lambda i: (0, 0)),
            pl.BlockSpec((D, D), lambda i: (0, 0)),
            pl.BlockSpec((1, D), lambda i: (0, 0)),
            pl.BlockSpec((BLK, RD), _rmap),
            pl.BlockSpec((RD, D), lambda i: (0, 0)),
        ],
        out_specs=[
            pl.BlockSpec((BLK, D), lambda i: (i, 0)),
            pl.BlockSpec((BLK, D), _nmap),
        ],
        out_shape=[
            jax.ShapeDtypeStruct((TAB_ROWS, D), jnp.float32),
            jax.ShapeDtypeStruct((NP, D), jnp.float32),
        ],
    )(s, acc, cnt, g2, b2, wn, ws, bs2, relemb, wr)


def _tc_post_body(s_ref, acc_ref, cnt_ref, g_ref, b_ref, o_ref):
    agg = acc_ref[0, :, :] + acc_ref[1, :, :]
    agg = agg / jnp.maximum(cnt_ref[...], 1.0)
    o_ref[...] = _ln_relu(s_ref[...] + agg, g_ref[...], b_ref[...])


def _tc_post(s, acc, cnt, g2, b2):
    return pl.pallas_call(
        _tc_post_body,
        grid=(GRID_N,),
        in_specs=[
            pl.BlockSpec((BLK, D), lambda i: (i, 0)),
            pl.BlockSpec((2, BLK, D), lambda i: (0, i, 0)),
            pl.BlockSpec((BLK, 1), lambda i: (i, 0)),
            pl.BlockSpec((1, D), lambda i: (0, 0)),
            pl.BlockSpec((1, D), lambda i: (0, 0)),
        ],
        out_specs=pl.BlockSpec((BLK, D), lambda i: (i, 0)),
        out_shape=jax.ShapeDtypeStruct((NP, D), jnp.float32),
    )(s, acc, cnt, g2, b2)


@functools.cache
def _mesh():
    return plsc.VectorSubcoreMesh(core_axis_name="c", subcore_axis_name="s",
                                  num_cores=NC, num_subcores=NS)


def _sc_edge_body(with_cnt, tab_hbm, idx_hbm, acc_out, cnt_out,
                  idx0, idx1, idx2, idx3, bg0, bg1, bg2, bg3,
                  sx0, sx1, sx2, sx3, ones_v, zb_v, acc_s, cnt_s,
                  si0, si1, si2, si3, sg0, sg1, sg2, sg3,
                  ss0, ss1, ss2, ss3):
    cid = lax.axis_index("c")
    sid = lax.axis_index("s")
    w = cid * NS + sid

    zero16 = jnp.zeros((16,), jnp.float32)
    one16 = jnp.ones((16,), jnp.float32)

    def _fill(i, _):
        for c in range(D // 16):
            bg0[i, pl.ds(c * 16, 16)] = zero16
        return 0

    lax.fori_loop(0, ROWS, _fill, 0)

    for i in range(ROWS // 16):
        ones_v[pl.ds(i * 16, 16)] = one16

    def _fillz(i, _):
        zb_v[pl.ds(i * 16, 16)] = zero16
        return 0

    lax.fori_loop(0, ZROWS // 16, _fillz, 0)

    zbase = sid * ZROWS
    for k in range(ZROWS // ROWS):
        pltpu.sync_copy(bg0.at[pl.ds(0, ROWS)],
                        acc_s.at[pl.ds(zbase + k * ROWS, ROWS)])
    if with_cnt:
        pltpu.sync_copy(zb_v, cnt_s.at[pl.ds(zbase, ZROWS)])

    plsc.subcore_barrier()

    idx = (idx0, idx1, idx2, idx3)
    bg = (bg0, bg1, bg2, bg3)
    sx = (sx0, sx1, sx2, sx3)
    sem_i = (si0, si1, si2, si3)
    sem_g = (sg0, sg1, sg2, sg3)
    sem_s = (ss0, ss1, ss2, ss3)

    pltpu.sync_copy(idx_hbm.at[w, 0], idx0)
    pltpu.sync_copy(idx_hbm.at[w, 1], idx1)
    pltpu.async_copy(tab_hbm.at[idx0.at[0]], bg0, sg0)
    pltpu.async_copy(tab_hbm.at[idx1.at[0]], bg1, sg1)
    pltpu.async_copy(idx_hbm.at[w, 2], idx2, si2)
    pltpu.async_copy(idx_hbm.at[w, 3], idx3, si3)

    def _quad(gi, _):
        for p in range(NB):
            j = NB * gi + p
            q = (p + 2) % NB

            @pl.when(j + 2 < CH)
            def _():
                pltpu.make_async_copy(idx_hbm.at[w, 0], idx[q],
                                      sem_i[q]).wait()

                @pl.when(j >= 2)
                def _():
                    pltpu.make_async_copy(
                        bg[q], acc_s.at[sx[q].at[0]], sem_s[q]).wait()

                pltpu.async_copy(tab_hbm.at[idx[q].at[0]], bg[q], sem_g[q])

            for k in range(ROWS // 16):
                sx[p][0, pl.ds(k * 16, 16)] = idx[p][1, pl.ds(k * 16, 16)]

            pltpu.make_async_copy(tab_hbm.at[idx[p].at[0]], bg[p],
                                  sem_g[p]).wait()
            pltpu.async_copy(bg[p], acc_s.at[sx[p].at[0]], sem_s[p],
                             add=True)
            if with_cnt:
                pltpu.sync_copy(ones_v, cnt_s.at[idx[p].at[2]], add=True)

            @pl.when(j + NB < CH)
            def _():
                pltpu.async_copy(idx_hbm.at[w, j + NB], idx[p], sem_i[p])

        return 0

    lax.fori_loop(0, CH // NB, _quad, 0)

    for p in range(NB):
        pltpu.make_async_copy(bg[p], acc_s.at[sx[p].at[0]], sem_s[p]).wait()

    plsc.subcore_barrier()

    pltpu.sync_copy(acc_s.at[pl.ds(zbase, ZROWS)],
                    acc_out.at[cid, pl.ds(zbase, ZROWS)])
    if with_cnt:
        pltpu.sync_copy(cnt_s.at[pl.ds(zbase, ZROWS)],
                        cnt_out.at[cid, pl.ds(zbase, ZROWS)])


@functools.cache
def _sc_edge(with_cnt):
    return functools.partial(
        pl.kernel,
        out_type=[
            jax.ShapeDtypeStruct((NC, NP, D), jnp.float32),
            jax.ShapeDtypeStruct((NC, NP), jnp.float32),
        ],
        mesh=_mesh(),
        compiler_params=pltpu.CompilerParams(needs_layout_passes=False),
        scratch_types=(
            [pltpu.VMEM((3, ROWS), jnp.int32)] * NB
            + [pltpu.VMEM((ROWS, D), jnp.float32)] * NB
            + [pltpu.VMEM((1, ROWS), jnp.int32)] * NB
            + [
                pltpu.VMEM((ROWS,), jnp.float32),
                pltpu.VMEM((ZROWS,), jnp.float32),
                pltpu.VMEM_SHARED((NP, D), jnp.float32),
                pltpu.VMEM_SHARED((NP,), jnp.float32),
            ]
            + [pltpu.SemaphoreType.DMA] * (3 * NB)
        ),
    )(functools.partial(_sc_edge_body, with_cnt))


def kernel(x, label, edge_index, edge_rel, label_emb, Win, bin_, rel_emb0,
           Wn0, Ws0, bs0, Wr0, g0, beta0, rel_emb1, Wn1, Ws1, bs1, Wr1, g1,
           beta1):
    wtop = Win[:D]
    wbot = Win[D:]
    bin2 = bin_.reshape(1, D)
    bs02 = bs0.reshape(1, D)
    bs12 = bs1.reshape(1, D)
    g02 = g0.reshape(1, D)
    b02 = beta0.reshape(1, D)
    g12 = g1.reshape(1, D)
    b12 = beta1.reshape(1, D)

    xp = jnp.concatenate([x, jnp.zeros((NP - N, D), jnp.float32)])
    lab = jnp.concatenate(
        [label.astype(jnp.int32), jnp.zeros((NP - N,), jnp.int32)]
    ).reshape(NP, 1)

    pad = jnp.arange(EPAD - E, dtype=jnp.int32)
    src = jnp.concatenate(
        [edge_index[0].astype(jnp.int32), pad % N]).reshape(NW, CH, CHUNK)
    dst = jnp.concatenate(
        [edge_index[1].astype(jnp.int32), N + pad % (NP - N)]
    ).reshape(NW, CH, CHUNK)
    rel = (NP + jnp.concatenate(
        [edge_rel.astype(jnp.int32), pad % RB])).reshape(NW, CH, CHUNK)
    trash = (N + jnp.arange(EPAD, dtype=jnp.int32) % (NP - N)
             ).reshape(NW, CH, CHUNK)
    gath = jnp.concatenate([src, rel], axis=-1)
    scat = jnp.concatenate([dst, dst], axis=-1)
    cntr = jnp.concatenate([dst, trash], axis=-1)
    eidx = jnp.stack([gath, scat, cntr], axis=2)

    tab0, s0 = _tc_first(xp, lab, label_emb, wtop, wbot, bin2,
                         Wn0, Ws0, bs02, rel_emb0, Wr0)
    acc0, cntp = _sc_edge(True)(tab0, eidx)
    cnt = (cntp[0] + cntp[1]).reshape(NP, 1)
    tab1, s1 = _tc_mid(s0, acc0, cnt, g02, b02, Wn1, Ws1, bs12,
                       rel_emb1, Wr1)
    acc1, _ = _sc_edge(False)(tab1, eidx)
    return _tc_post(s1, acc1, cnt, g12, b12)[:N]


# speedup vs baseline: 1.0316x; 1.0316x over previous
import functools

import jax
import jax.numpy as jnp
from jax import lax
from jax.experimental import pallas as pl
from jax.experimental.pallas import tpu as pltpu
from jax.experimental.pallas import tpu_sc as plsc

N = 10000
E = 320000
D = 128
LD = 32
RD = 32
NL = 1024
RB = 4096

NC = 2
NS = 16
NW = NC * NS

NP = 10240
TAB_ROWS = NP + RB
CHUNK = 64
ROWS = 2 * CHUNK
CH = 160
EPAD = NW * CH * CHUNK
ZROWS = NP // NS

BLK = 2048
GRID_N = NP // BLK
GRID_TAB = TAB_ROWS // BLK


def _ln_relu(pre, g, b):
    h = jnp.maximum(pre, 0.0)
    mu = jnp.mean(h, axis=-1, keepdims=True)
    var = jnp.mean((h - mu) * (h - mu), axis=-1, keepdims=True)
    return (h - mu) * lax.rsqrt(var + 1e-5) * g + b


def _nmap(i):
    return (jnp.minimum(i, GRID_N - 1), 0)


def _rmap(i):
    return (jnp.maximum(i - GRID_N, 0), 0)


def _tc_first_body(x_ref, lab_ref, lemb_ref, wtop_ref, wbot_ref, bin_ref,
                   wn_ref, ws_ref, bs_ref, rel_ref, wr_ref,
                   tab_ref, s_ref, t_ref):
    i = pl.program_id(0)

    @pl.when(i == 0)
    def _():
        t_ref[...] = lemb_ref[...] @ wbot_ref[...]

    @pl.when(i < GRID_N)
    def _():
        onehot = jnp.where(
            lab_ref[...] == lax.broadcasted_iota(jnp.int32, (1, NL), 1),
            1.0, 0.0)
        h = jnp.maximum(
            x_ref[...] @ wtop_ref[...] + onehot @ t_ref[...] + bin_ref[...],
            0.0)
        tab_ref[...] = h @ wn_ref[...]
        s_ref[...] = h @ ws_ref[...] + bs_ref[...]

    @pl.when(i >= GRID_N)
    def _():
        tab_ref[...] = rel_ref[...] @ wr_ref[...]


def _tc_first(x, lab, lemb, wtop, wbot, bin2, wn, ws, bs2, relemb, wr):
    return pl.pallas_call(
        _tc_first_body,
        grid=(GRID_TAB,),
        in_specs=[
            pl.BlockSpec((BLK, D), _nmap),
            pl.BlockSpec((BLK, 1), _nmap),
            pl.BlockSpec((NL, LD), lambda i: (0, 0)),
            pl.BlockSpec((D, D), lambda i: (0, 0)),
            pl.BlockSpec((LD, D), lambda i: (0, 0)),
            pl.BlockSpec((1, D), lambda i: (0, 0)),
            pl.BlockSpec((D, D), lambda i: (0, 0)),
            pl.BlockSpec((D, D), lambda i: (0, 0)),
            pl.BlockSpec((1, D), lambda i: (0, 0)),
            pl.BlockSpec((BLK, RD), _rmap),
            pl.BlockSpec((RD, D), lambda i: (0, 0)),
        ],
        out_specs=[
            pl.BlockSpec((BLK, D), lambda i: (i, 0)),
            pl.BlockSpec((BLK, D), _nmap),
        ],
        out_shape=[
            jax.ShapeDtypeStruct((TAB_ROWS, D), jnp.float32),
            jax.ShapeDtypeStruct((NP, D), jnp.float32),
        ],
        scratch_shapes=[pltpu.VMEM((NL, D), jnp.float32)],
    )(x, lab, lemb, wtop, wbot, bin2, wn, ws, bs2, relemb, wr)


def _tc_mid_body(s_ref, acc_ref, cnt_ref, g_ref, b_ref, wn_ref, ws_ref,
                 bs_ref, rel_ref, wr_ref, tab_ref, so_ref):
    i = pl.program_id(0)

    @pl.when(i < GRID_N)
    def _():
        agg = acc_ref[0, :, :] + acc_ref[1, :, :]
        agg = agg / jnp.maximum(cnt_ref[...], 1.0)
        h = _ln_relu(s_ref[...] + agg, g_ref[...], b_ref[...])
        tab_ref[...] = h @ wn_ref[...]
        so_ref[...] = h @ ws_ref[...] + bs_ref[...]

    @pl.when(i >= GRID_N)
    def _():
        tab_ref[...] = rel_ref[...] @ wr_ref[...]


def _tc_mid(s, acc, cnt, g2, b2, wn, ws, bs2, relemb, wr):
    return pl.pallas_call(
        _tc_mid_body,
        grid=(GRID_TAB,),
        in_specs=[
            pl.BlockSpec((BLK, D), _nmap),
            pl.BlockSpec((2, BLK, D), lambda i: (0, *_nmap(i))),
            pl.BlockSpec((BLK, 1), _nmap),
            pl.BlockSpec((1, D), lambda i: (0, 0)),
            pl.BlockSpec((1, D), lambda i: (0, 0)),
            pl.BlockSpec((D, D), lambda i: (0, 0)),
            pl.BlockSpec((D, D), lambda i: (0, 0)),
            pl.BlockSpec((1, D), lambda i: (0, 0)),
            pl.BlockSpec((BLK, RD), _rmap),
            pl.BlockSpec((RD, D), lambda i: (0, 0)),
        ],
        out_specs=[
            pl.BlockSpec((BLK, D), lambda i: (i, 0)),
            pl.BlockSpec((BLK, D), _nmap),
        ],
        out_shape=[
            jax.ShapeDtypeStruct((TAB_ROWS, D), jnp.float32),
            jax.ShapeDtypeStruct((NP, D), jnp.float32),
        ],
    )(s, acc, cnt, g2, b2, wn, ws, bs2, relemb, wr)


def _tc_post_body(s_ref, acc_ref, cnt_ref, g_ref, b_ref, o_ref):
    agg = acc_ref[0, :, :] + acc_ref[1, :, :]
    agg = agg / jnp.maximum(cnt_ref[...], 1.0)
    o_ref[...] = _ln_relu(s_ref[...] + agg, g_ref[...], b_ref[...])


def _tc_post(s, acc, cnt, g2, b2):
    return pl.pallas_call(
        _tc_post_body,
        grid=(GRID_N,),
        in_specs=[
            pl.BlockSpec((BLK, D), lambda i: (i, 0)),
            pl.BlockSpec((2, BLK, D), lambda i: (0, i, 0)),
            pl.BlockSpec((BLK, 1), lambda i: (i, 0)),
            pl.BlockSpec((1, D), lambda i: (0, 0)),
            pl.BlockSpec((1, D), lambda i: (0, 0)),
        ],
        out_specs=pl.BlockSpec((BLK, D), lambda i: (i, 0)),
        out_shape=jax.ShapeDtypeStruct((NP, D), jnp.float32),
    )(s, acc, cnt, g2, b2)


@functools.cache
def _mesh():
    return plsc.VectorSubcoreMesh(core_axis_name="c", subcore_axis_name="s",
                                  num_cores=NC, num_subcores=NS)


def _sc_edge_body(with_cnt, tab_hbm, idx_hbm, acc_out, cnt_out,
                  idx0, idx1, bg0, bg1, sx0, sx1, ones_v, zb_v,
                  acc_s, cnt_s, sem_i, sem_g0, sem_g1, sem_s0, sem_s1):
    cid = lax.axis_index("c")
    sid = lax.axis_index("s")
    w = cid * NS + sid

    zero16 = jnp.zeros((16,), jnp.float32)
    one16 = jnp.ones((16,), jnp.float32)

    def _fill(i, _):
        for c in range(D // 16):
            bg0[i, pl.ds(c * 16, 16)] = zero16
        return 0

    lax.fori_loop(0, ROWS, _fill, 0)

    for i in range(ROWS // 16):
        ones_v[pl.ds(i * 16, 16)] = one16

    def _fillz(i, _):
        zb_v[pl.ds(i * 16, 16)] = zero16
        return 0

    lax.fori_loop(0, ZROWS // 16, _fillz, 0)

    zbase = sid * ZROWS
    for k in range(ZROWS // ROWS):
        pltpu.sync_copy(bg0.at[pl.ds(0, ROWS)],
                        acc_s.at[pl.ds(zbase + k * ROWS, ROWS)])
    if with_cnt:
        pltpu.sync_copy(zb_v, cnt_s.at[pl.ds(zbase, ZROWS)])

    plsc.subcore_barrier()

    idx = (idx0, idx1)
    bg = (bg0, bg1)
    sx = (sx0, sx1)
    sem_g = (sem_g0, sem_g1)
    sem_s = (sem_s0, sem_s1)

    pltpu.sync_copy(idx_hbm.at[w, 0], idx0)
    pltpu.async_copy(tab_hbm.at[idx0.at[0]], bg0, sem_g0)
    pltpu.async_copy(idx_hbm.at[w, 1], idx1, sem_i)

    def _pair(gi, _):
        for p in (0, 1):
            j = 2 * gi + p
            q = 1 - p

            @pl.when(j + 1 < CH)
            def _():
                pltpu.make_async_copy(idx_hbm.at[w, 0], idx[q], sem_i).wait()

                @pl.when(j >= 1)
                def _():
                    pltpu.make_async_copy(
                        bg[q], acc_s.at[sx[q].at[0]], sem_s[q]).wait()

                pltpu.async_copy(tab_hbm.at[idx[q].at[0]], bg[q], sem_g[q])

            for k in range(ROWS // 16):
                sx[p][0, pl.ds(k * 16, 16)] = idx[p][1, pl.ds(k * 16, 16)]

            pltpu.make_async_copy(tab_hbm.at[idx[p].at[0]], bg[p],
                                  sem_g[p]).wait()
            pltpu.async_copy(bg[p], acc_s.at[sx[p].at[0]], sem_s[p],
                             add=True)
            if with_cnt:
                pltpu.sync_copy(ones_v, cnt_s.at[idx[p].at[2]], add=True)

            @pl.when(j + 2 < CH)
            def _():
                pltpu.async_copy(idx_hbm.at[w, j + 2], idx[p], sem_i)

        return 0

    lax.fori_loop(0, CH // 2, _pair, 0)

    for p in (0, 1):
        pltpu.make_async_copy(bg[p], acc_s.at[sx[p].at[0]], sem_s[p]).wait()

    plsc.subcore_barrier()

    pltpu.sync_copy(acc_s.at[pl.ds(zbase, ZROWS)],
                    acc_out.at[cid, pl.ds(zbase, ZROWS)])
    if with_cnt:
        pltpu.sync_copy(cnt_s.at[pl.ds(zbase, ZROWS)],
                        cnt_out.at[cid, pl.ds(zbase, ZROWS)])


@functools.cache
def _sc_edge(with_cnt):
    return functools.partial(
        pl.kernel,
        out_type=[
            jax.ShapeDtypeStruct((NC, NP, D), jnp.float32),
            jax.ShapeDtypeStruct((NC, NP), jnp.float32),
        ],
        mesh=_mesh(),
        compiler_params=pltpu.CompilerParams(needs_layout_passes=False),
        scratch_types=[
            pltpu.VMEM((3, ROWS), jnp.int32),
            pltpu.VMEM((3, ROWS), jnp.int32),
            pltpu.VMEM((ROWS, D), jnp.float32),
            pltpu.VMEM((ROWS, D), jnp.float32),
            pltpu.VMEM((1, ROWS), jnp.int32),
            pltpu.VMEM((1, ROWS), jnp.int32),
            pltpu.VMEM((ROWS,), jnp.float32),
            pltpu.VMEM((ZROWS,), jnp.float32),
            pltpu.VMEM_SHARED((NP, D), jnp.float32),
            pltpu.VMEM_SHARED((NP,), jnp.float32),
            pltpu.SemaphoreType.DMA,
            pltpu.SemaphoreType.DMA,
            pltpu.SemaphoreType.DMA,
            pltpu.SemaphoreType.DMA,
            pltpu.SemaphoreType.DMA,
        ],
    )(functools.partial(_sc_edge_body, with_cnt))


def kernel(x, label, edge_index, edge_rel, label_emb, Win, bin_, rel_emb0,
           Wn0, Ws0, bs0, Wr0, g0, beta0, rel_emb1, Wn1, Ws1, bs1, Wr1, g1,
           beta1):
    wtop = Win[:D]
    wbot = Win[D:]
    bin2 = bin_.reshape(1, D)
    bs02 = bs0.reshape(1, D)
    bs12 = bs1.reshape(1, D)
    g02 = g0.reshape(1, D)
    b02 = beta0.reshape(1, D)
    g12 = g1.reshape(1, D)
    b12 = beta1.reshape(1, D)

    xp = jnp.concatenate([x, jnp.zeros((NP - N, D), jnp.float32)])
    lab = jnp.concatenate(
        [label.astype(jnp.int32), jnp.zeros((NP - N,), jnp.int32)]
    ).reshape(NP, 1)

    pad = jnp.arange(EPAD - E, dtype=jnp.int32)
    src = jnp.concatenate(
        [edge_index[0].astype(jnp.int32), pad % N]).reshape(NW, CH, CHUNK)
    dst = jnp.concatenate(
        [edge_index[1].astype(jnp.int32), N + pad % (NP - N)]
    ).reshape(NW, CH, CHUNK)
    rel = (NP + jnp.concatenate(
        [edge_rel.astype(jnp.int32), pad % RB])).reshape(NW, CH, CHUNK)
    trash = (N + jnp.arange(EPAD, dtype=jnp.int32) % (NP - N)
             ).reshape(NW, CH, CHUNK)
    gath = jnp.concatenate([src, rel], axis=-1)
    scat = jnp.concatenate([dst, dst], axis=-1)
    cntr = jnp.concatenate([dst, trash], axis=-1)
    eidx = jnp.stack([gath, scat, cntr], axis=2)

    tab0, s0 = _tc_first(xp, lab, label_emb, wtop, wbot, bin2,
                         Wn0, Ws0, bs02, rel_emb0, Wr0)
    acc0, cntp = _sc_edge(True)(tab0, eidx)
    cnt = (cntp[0] + cntp[1]).reshape(NP, 1)
    tab1, s1 = _tc_mid(s0, acc0, cnt, g02, b02, Wn1, Ws1, bs12,
                       rel_emb1, Wr1)
    acc1, _ = _sc_edge(False)(tab1, eidx)
    return _tc_post(s1, acc1, cnt, g12, b12)[:N]
